# baseline (speedup 1.0000x reference)
import functools

import jax
import jax.numpy as jnp
from jax import lax
from jax.experimental import pallas as pl
from jax.experimental.pallas import tpu as pltpu
from jax.experimental.pallas import tpu_sc as plsc

DIM = 64


def _gather_kernel(idx_hbm, table_hbm, out_hbm, idx_v, rows_v, sem,
                   *, num_cores, b_per_w, chunk, n_chunks):
    wid = lax.axis_index("s") * num_cores + lax.axis_index("c")
    base = wid * b_per_w

    def step(g, carry):
        off = base + g * chunk
        pltpu.sync_copy(idx_hbm.at[pl.ds(off, chunk)], idx_v)
        pltpu.async_copy(table_hbm.at[idx_v], rows_v, sem).wait()
        pltpu.sync_copy(rows_v, out_hbm.at[pl.ds(off, chunk)])
        return carry

    lax.fori_loop(0, n_chunks, step, 0)


def kernel(idxes, relEmbbed):
    b0, b1 = idxes.shape
    total = b0 * b1
    info = plsc.get_sparse_core_info()
    num_workers = info.num_cores * info.num_subcores
    assert total % num_workers == 0
    b_per_w = total // num_workers
    chunk = 832
    assert b_per_w % chunk == 0
    n_chunks = b_per_w // chunk

    mesh = plsc.VectorSubcoreMesh(core_axis_name="c", subcore_axis_name="s")
    body = functools.partial(
        _gather_kernel,
        num_cores=info.num_cores,
        b_per_w=b_per_w,
        chunk=chunk,
        n_chunks=n_chunks,
    )
    run = pl.kernel(
        body,
        mesh=mesh,
        compiler_params=pltpu.CompilerParams(use_tc_tiling_on_sc=False),
        out_type=jax.ShapeDtypeStruct((total, DIM), jnp.float32),
        scratch_types=[
            pltpu.VMEM((chunk,), jnp.int32),
            pltpu.VMEM((chunk, DIM), jnp.float32),
            pltpu.SemaphoreType.DMA,
        ],
    )
    flat_idx = idxes.reshape(total).astype(jnp.int32)
    out = run(flat_idx, relEmbbed)
    return out.reshape(b0, b1, DIM)


# speedup vs baseline: 1.0121x; 1.0121x over previous
import functools

import jax
import jax.numpy as jnp
from jax import lax
from jax.experimental import pallas as pl
from jax.experimental.pallas import tpu as pltpu
from jax.experimental.pallas import tpu_sc as plsc

DIM = 64
CHUNK = 832
NBUF = 2


def _gather_kernel(idx_hbm, table_hbm, out_hbm, idx_v, rows, g_sems, o_sems,
                   *, num_cores, b_per_w, n_chunks):
    wid = lax.axis_index("s") * num_cores + lax.axis_index("c")
    base = wid * b_per_w

    pltpu.sync_copy(idx_hbm.at[wid], idx_v)

    def gather_start(g):
        s = g % NBUF
        return pltpu.async_copy(table_hbm.at[idx_v.at[g]], rows[s], g_sems[s])

    gathers = [None] * n_chunks
    writes = [None] * n_chunks
    gathers[0] = gather_start(0)
    for g in range(n_chunks):
        s = g % NBUF
        if g + 1 < n_chunks:
            sn = (g + 1) % NBUF
            if g + 1 >= NBUF:
                writes[g + 1 - NBUF].wait()
            gathers[g + 1] = gather_start(g + 1)
        gathers[g].wait()
        writes[g] = pltpu.async_copy(
            rows[s], out_hbm.at[pl.ds(base + g * CHUNK, CHUNK)], o_sems[s])
    for g in range(max(0, n_chunks - NBUF), n_chunks):
        writes[g].wait()


def kernel(idxes, relEmbbed):
    b0, b1 = idxes.shape
    total = b0 * b1
    info = plsc.get_sparse_core_info()
    num_workers = info.num_cores * info.num_subcores
    assert total % num_workers == 0
    b_per_w = total // num_workers
    assert b_per_w % CHUNK == 0
    n_chunks = b_per_w // CHUNK

    mesh = plsc.VectorSubcoreMesh(core_axis_name="c", subcore_axis_name="s")
    body = functools.partial(
        _gather_kernel,
        num_cores=info.num_cores,
        b_per_w=b_per_w,
        n_chunks=n_chunks,
    )
    run = pl.kernel(
        body,
        mesh=mesh,
        compiler_params=pltpu.CompilerParams(use_tc_tiling_on_sc=False),
        out_type=jax.ShapeDtypeStruct((total, DIM), jnp.float32),
        scratch_types=[
            pltpu.VMEM((n_chunks, CHUNK), jnp.int32),
            [pltpu.VMEM((CHUNK, DIM), jnp.float32) for _ in range(NBUF)],
            [pltpu.SemaphoreType.DMA for _ in range(NBUF)],
            [pltpu.SemaphoreType.DMA for _ in range(NBUF)],
        ],
    )
    flat_idx = idxes.reshape(num_workers, n_chunks, CHUNK).astype(jnp.int32)
    out = run(flat_idx, relEmbbed)
    return out.reshape(b0, b1, DIM)


# speedup vs baseline: 1.0558x; 1.0432x over previous
import functools

import jax
import jax.numpy as jnp
from jax import lax
from jax.experimental import pallas as pl
from jax.experimental.pallas import tpu as pltpu
from jax.experimental.pallas import tpu_sc as plsc

DIM = 64
CHUNK = 512
NBUF = 3


def _gather_kernel(idx_hbm, table_hbm, out_hbm, idx_v, rows, g_sems, o_sems,
                   *, num_cores, b_per_w, n_chunks, n_b0):
    wid = lax.axis_index("s") * num_cores + lax.axis_index("c")
    base = wid * b_per_w

    pltpu.sync_copy(idx_hbm.at[wid], idx_v)

    def gather_start(g):
        s = g % NBUF
        return pltpu.async_copy(table_hbm.at[idx_v.at[g]], rows[s], g_sems[s])

    def write_start(g):
        s = g % NBUF
        c0 = base + g * CHUNK
        b1c = c0 // n_b0
        b0c = c0 % n_b0
        return pltpu.async_copy(
            rows[s], out_hbm.at[b1c, pl.ds(b0c, CHUNK)], o_sems[s])

    gathers = [None] * n_chunks
    writes = [None] * n_chunks
    gathers[0] = gather_start(0)
    for g in range(n_chunks):
        if g + 1 < n_chunks:
            if g + 1 >= NBUF:
                writes[g + 1 - NBUF].wait()
            gathers[g + 1] = gather_start(g + 1)
        gathers[g].wait()
        writes[g] = write_start(g)
    for g in range(max(0, n_chunks - NBUF), n_chunks):
        writes[g].wait()


def kernel(idxes, relEmbbed):
    b0, b1 = idxes.shape
    total = b0 * b1
    info = plsc.get_sparse_core_info()
    num_workers = info.num_cores * info.num_subcores
    assert total % num_workers == 0
    b_per_w = total // num_workers
    assert b_per_w % CHUNK == 0 and b0 % CHUNK == 0
    n_chunks = b_per_w // CHUNK

    mesh = plsc.VectorSubcoreMesh(core_axis_name="c", subcore_axis_name="s")
    body = functools.partial(
        _gather_kernel,
        num_cores=info.num_cores,
        b_per_w=b_per_w,
        n_chunks=n_chunks,
        n_b0=b0,
    )
    run = pl.kernel(
        body,
        mesh=mesh,
        compiler_params=pltpu.CompilerParams(use_tc_tiling_on_sc=False),
        out_type=jax.ShapeDtypeStruct((b1, b0, DIM), jnp.float32),
        scratch_types=[
            pltpu.VMEM((n_chunks, CHUNK), jnp.int32),
            [pltpu.VMEM((CHUNK, DIM), jnp.float32) for _ in range(NBUF)],
            [pltpu.SemaphoreType.DMA for _ in range(NBUF)],
            [pltpu.SemaphoreType.DMA for _ in range(NBUF)],
        ],
    )
    flat_idx = idxes.T.reshape(num_workers, n_chunks, CHUNK).astype(jnp.int32)
    out3 = run(flat_idx, relEmbbed)
    return out3.transpose(1, 0, 2)
